# Initial kernel scaffold; baseline (speedup 1.0000x reference)
import jax
import jax.numpy as jnp
from jax.experimental import pallas as pl


def kernel(x, edge_index, batch, W1, b1, W2, b2, W3, b3, W4, b4, Wf, bf):
    raise NotImplementedError("write your pallas kernel here")


# speedup vs baseline: 33.9437x; 33.9437x over previous
import functools

import jax
import jax.numpy as jnp
import numpy as np
from jax import lax
from jax.experimental import pallas as pl
from jax.experimental.pallas import tpu as pltpu
from jax.experimental.pallas import tpu_sc as plsc

N = 10000
E = 320000
F = 128
NT = 10112
D = 16
CB = 128
NC, NS = 2, 16
NW = NC * NS
T_CH = 80
EP = NW * T_CH * CB
RS = NT // 8
NROW = N // 8
STRIPE = NT // NS


def _sc_mesh():
    return plsc.VectorSubcoreMesh(core_axis_name="c", subcore_axis_name="s")


@functools.partial(
    pl.kernel,
    out_type=jax.ShapeDtypeStruct((NC, NT, D), jnp.float32),
    mesh=_sc_mesh(),
    compiler_params=pltpu.CompilerParams(use_tc_tiling_on_sc=False),
    scratch_types=[
        pltpu.VMEM((T_CH, CB), jnp.int32),
        pltpu.VMEM((T_CH, CB), jnp.int32),
        pltpu.VMEM((CB, D), jnp.float32),
        pltpu.VMEM_SHARED((NT, D), jnp.float32),
        pltpu.SemaphoreType.DMA,
    ],
)
def _sc_agg(tab_hbm, src_hbm, dst_hbm, zeros_hbm, out_hbm,
            sidx, didx, rows, acc, sem):
    c = lax.axis_index("c")
    s = lax.axis_index("s")
    wid = c * NS + s
    pltpu.sync_copy(zeros_hbm.at[pl.ds(s * STRIPE, STRIPE)],
                    acc.at[pl.ds(s * STRIPE, STRIPE)])
    pltpu.sync_copy(src_hbm.at[pl.ds(wid * T_CH, T_CH)], sidx)
    pltpu.sync_copy(dst_hbm.at[pl.ds(wid * T_CH, T_CH)], didx)
    plsc.subcore_barrier()

    def body(i, carry):
        pltpu.async_copy(tab_hbm.at[sidx.at[i]], rows, sem).wait()
        pltpu.sync_copy(rows, acc.at[didx.at[i]], add=True)
        return carry

    lax.fori_loop(0, T_CH, body, 0)
    plsc.subcore_barrier()
    pltpu.sync_copy(acc.at[pl.ds(s * STRIPE, STRIPE)],
                    out_hbm.at[c, pl.ds(s * STRIPE, STRIPE)])


@functools.partial(
    pl.kernel,
    out_type=jax.ShapeDtypeStruct((NC, NT, D), jnp.float32),
    mesh=_sc_mesh(),
    compiler_params=pltpu.CompilerParams(use_tc_tiling_on_sc=False),
    scratch_types=[
        pltpu.VMEM((T_CH, CB), jnp.int32),
        pltpu.VMEM((CB, D), jnp.float32),
        pltpu.VMEM_SHARED((NT, D), jnp.float32),
    ],
)
def _sc_deg(ones_hbm, dst_hbm, zeros_hbm, out_hbm, didx, rows, acc):
    c = lax.axis_index("c")
    s = lax.axis_index("s")
    wid = c * NS + s
    pltpu.sync_copy(zeros_hbm.at[pl.ds(s * STRIPE, STRIPE)],
                    acc.at[pl.ds(s * STRIPE, STRIPE)])
    pltpu.sync_copy(dst_hbm.at[pl.ds(wid * T_CH, T_CH)], didx)
    pltpu.sync_copy(ones_hbm, rows)
    plsc.subcore_barrier()

    def body(i, carry):
        pltpu.sync_copy(rows, acc.at[didx.at[i]], add=True)
        return carry

    lax.fori_loop(0, T_CH, body, 0)
    plsc.subcore_barrier()
    pltpu.sync_copy(acc.at[pl.ds(s * STRIPE, STRIPE)],
                    out_hbm.at[c, pl.ds(s * STRIPE, STRIPE)])


def _tc0_body(x_ref, w_ref, deg_ref, u_ref, dinv_ref):
    deg = deg_ref[0] + deg_ref[1] + 1.0
    rmask = lax.broadcasted_iota(jnp.int32, (RS, 128), 0) < NROW
    dinv = jnp.where(rmask, lax.rsqrt(deg), 0.0)
    z = jnp.dot(x_ref[...], w_ref[...], preferred_element_type=jnp.float32)
    u_ref[...] = dinv * z
    dinv_ref[...] = dinv


_tc0 = pl.pallas_call(
    _tc0_body,
    out_shape=(
        jax.ShapeDtypeStruct((RS, 128), jnp.float32),
        jax.ShapeDtypeStruct((RS, 128), jnp.float32),
    ),
)


def _tc1_body(agg_ref, t_ref, dinv_ref, b_ref, out_ref):
    dinv = dinv_ref[...]
    m = dinv * (agg_ref[0] + agg_ref[1] + t_ref[...])
    h = jnp.maximum(m + b_ref[...], 0.0)
    out_ref[...] = dinv * h


_tc1 = pl.pallas_call(
    _tc1_body,
    out_shape=jax.ShapeDtypeStruct((RS, 128), jnp.float32),
)


def _tcmid_body(agg_ref, t_ref, dinv_ref, w_ref, out_ref):
    dinv = dinv_ref[...]
    m = dinv * (agg_ref[0] + agg_ref[1] + t_ref[...])
    col = lax.broadcasted_iota(jnp.int32, (RS, 128), 1)
    m = jnp.where(col % D == 6, 1.0, m)
    h = jnp.maximum(
        jnp.dot(m, w_ref[...], preferred_element_type=jnp.float32), 0.0)
    out_ref[...] = dinv * h


_tcmid = pl.pallas_call(
    _tcmid_body,
    out_shape=jax.ShapeDtypeStruct((RS, 128), jnp.float32),
)


def _tcfin_body(agg_ref, t_ref, dinv_ref, w_ref, oh_ref, out_ref):
    dinv = dinv_ref[...]
    m = dinv * (agg_ref[0] + agg_ref[1] + t_ref[...])
    col = lax.broadcasted_iota(jnp.int32, (RS, 128), 1)
    m = jnp.where(col % D == 6, 1.0, m)
    h5 = jnp.maximum(
        jnp.dot(m, w_ref[...], preferred_element_type=jnp.float32), 0.0)
    oh = oh_ref[...]
    big = lax.dot_general(h5, oh, (((0,), (0,)), ((), ())),
                          preferred_element_type=jnp.float32)
    csum = jnp.sum(oh, axis=0, keepdims=True)
    sums = jnp.zeros((16, 16), jnp.float32)
    cnts = jnp.zeros((1, 16), jnp.float32)
    for k in range(8):
        sums = sums + big[k * 16:(k + 1) * 16, k * 16:(k + 1) * 16]
        cnts = cnts + csum[:, k * 16:(k + 1) * 16]
    mean_t = sums / jnp.maximum(cnts, 1.0)
    mx = jnp.max(mean_t, axis=0, keepdims=True)
    lse = jnp.log(jnp.sum(jnp.exp(mean_t - mx), axis=0, keepdims=True))
    out_ref[...] = mean_t - mx - lse


_tcfin = pl.pallas_call(
    _tcfin_body,
    out_shape=jax.ShapeDtypeStruct((16, 16), jnp.float32),
)


def _stack_w(W, b):
    Ws = jnp.zeros((D, D), jnp.float32)
    Ws = Ws.at[:W.shape[0], :W.shape[1]].set(W)
    return Ws.at[6, :b.shape[0]].set(b)


def _kron8(Ws):
    return jnp.kron(jnp.eye(8, dtype=jnp.float32), Ws)


def kernel(x, edge_index, batch, W1, b1, W2, b2, W3, b3, W4, b4, Wf, bf):
    f32 = jnp.float32
    src = edge_index[0]
    dst = edge_index[1]
    pad = EP - E
    pad_idx = N + (jnp.arange(pad, dtype=jnp.int32) % (NT - N))
    src2d = jnp.concatenate([src, pad_idx]).reshape(EP // CB, CB)
    dst2d = jnp.concatenate([dst, pad_idx]).reshape(EP // CB, CB)

    zeros_tab = jnp.zeros((NT, D), f32)
    ones_cb = jnp.ones((CB, D), f32)

    x_rs = jnp.zeros((NT, F), f32).at[:N].set(x).reshape(RS, 8 * F)
    w1big = jnp.kron(jnp.eye(8, dtype=f32),
                     jnp.pad(W1, ((0, 0), (0, D - W1.shape[1]))))
    b1bc = jnp.tile(jnp.pad(b1, (0, D - b1.shape[0])), 8)[None, :]

    batch_pad = jnp.concatenate(
        [batch, jnp.full((NT - N,), -1, jnp.int32)])
    oh_rs = (batch_pad[:, None] == jnp.arange(16)[None, :]).astype(
        f32).reshape(RS, 128)

    deg2 = _sc_deg(ones_cb, dst2d, zeros_tab)
    u1, dinv = _tc0(x_rs, w1big, deg2.reshape(NC, RS, 128))

    s = _sc_agg(u1.reshape(NT, D), src2d, dst2d, zeros_tab)
    t = _tc1(s.reshape(NC, RS, 128), u1, dinv, b1bc)

    for Wl, bl in ((W2, b2), (W3, b3), (W4, b4)):
        s = _sc_agg(t.reshape(NT, D), src2d, dst2d, zeros_tab)
        t = _tcmid(s.reshape(NC, RS, 128), t, dinv, _kron8(_stack_w(Wl, bl)))

    s = _sc_agg(t.reshape(NT, D), src2d, dst2d, zeros_tab)
    out_t = _tcfin(s.reshape(NC, RS, 128), t, dinv,
                   _kron8(_stack_w(Wf, bf)), oh_rs)
    return out_t.T


# speedup vs baseline: 58.6503x; 1.7279x over previous
import functools

import jax
import jax.numpy as jnp
import numpy as np
from jax import lax
from jax.experimental import pallas as pl
from jax.experimental.pallas import tpu as pltpu
from jax.experimental.pallas import tpu_sc as plsc

N = 10000
E = 320000
F = 128
NT = 10112
D = 16
CB = 128
NC, NS = 2, 16
NW = NC * NS
T_CH = 80
EP = NW * T_CH * CB
RS = NT // 8
NROW = N // 8
STRIPE = NT // NS


def _sc_mesh():
    return plsc.VectorSubcoreMesh(core_axis_name="c", subcore_axis_name="s")


NSLOT = 4


@functools.partial(
    pl.kernel,
    out_type=jax.ShapeDtypeStruct((NC, NT, D), jnp.float32),
    mesh=_sc_mesh(),
    compiler_params=pltpu.CompilerParams(use_tc_tiling_on_sc=False),
    scratch_types=[
        pltpu.VMEM((T_CH, CB), jnp.int32),
        pltpu.VMEM((T_CH, CB), jnp.int32),
        pltpu.VMEM((NSLOT, CB, D), jnp.float32),
        pltpu.VMEM_SHARED((NT, D), jnp.float32),
    ] + [pltpu.SemaphoreType.DMA] * (2 * NSLOT),
)
def _sc_agg(tab_hbm, src_hbm, dst_hbm, zeros_hbm, out_hbm,
            sidx, didx, rows, acc, *sems):
    gsem = sems[:NSLOT]
    ssem = sems[NSLOT:]
    c = lax.axis_index("c")
    s = lax.axis_index("s")
    wid = c * NS + s
    pltpu.sync_copy(zeros_hbm.at[pl.ds(s * STRIPE, STRIPE)],
                    acc.at[pl.ds(s * STRIPE, STRIPE)])
    pltpu.sync_copy(src_hbm.at[pl.ds(wid * T_CH, T_CH)], sidx)
    pltpu.sync_copy(dst_hbm.at[pl.ds(wid * T_CH, T_CH)], didx)
    plsc.subcore_barrier()

    def g_start(slot, i):
        pltpu.async_copy(tab_hbm.at[sidx.at[i]], rows.at[slot], gsem[slot])

    def g_wait(slot, i):
        pltpu.make_async_copy(
            tab_hbm.at[sidx.at[i]], rows.at[slot], gsem[slot]).wait()

    def s_start(slot, i):
        pltpu.async_copy(rows.at[slot], acc.at[didx.at[i]], ssem[slot],
                         add=True)

    def s_wait(slot, i):
        pltpu.make_async_copy(
            rows.at[slot], acc.at[didx.at[i]], ssem[slot]).wait()

    g_start(0, 0)
    g_start(1, 1)

    def body(k, carry):
        for b in range(NSLOT):
            i = k * NSLOT + b
            g_wait(b, i)
            s_start(b, i)
            nxt = (b + 2) % NSLOT

            @pl.when(i >= 2)
            def _():
                s_wait(nxt, i - 2)

            @pl.when(i + 2 < T_CH)
            def _():
                g_start(nxt, i + 2)
        return carry

    lax.fori_loop(0, T_CH // NSLOT, body, 0)
    s_wait((T_CH - 2) % NSLOT, T_CH - 2)
    s_wait((T_CH - 1) % NSLOT, T_CH - 1)
    plsc.subcore_barrier()
    pltpu.sync_copy(acc.at[pl.ds(s * STRIPE, STRIPE)],
                    out_hbm.at[c, pl.ds(s * STRIPE, STRIPE)])


@functools.partial(
    pl.kernel,
    out_type=jax.ShapeDtypeStruct((NC, NT, D), jnp.float32),
    mesh=_sc_mesh(),
    compiler_params=pltpu.CompilerParams(use_tc_tiling_on_sc=False),
    scratch_types=[
        pltpu.VMEM((T_CH, CB), jnp.int32),
        pltpu.VMEM((CB, D), jnp.float32),
        pltpu.VMEM_SHARED((NT, D), jnp.float32),
        pltpu.SemaphoreType.DMA,
    ],
)
def _sc_deg(ones_hbm, dst_hbm, zeros_hbm, out_hbm, didx, rows, acc, sem):
    c = lax.axis_index("c")
    s = lax.axis_index("s")
    wid = c * NS + s
    pltpu.sync_copy(zeros_hbm.at[pl.ds(s * STRIPE, STRIPE)],
                    acc.at[pl.ds(s * STRIPE, STRIPE)])
    pltpu.sync_copy(dst_hbm.at[pl.ds(wid * T_CH, T_CH)], didx)
    pltpu.sync_copy(ones_hbm, rows)
    plsc.subcore_barrier()

    def body(i, carry):
        pltpu.async_copy(rows, acc.at[didx.at[i]], sem, add=True)
        return carry

    lax.fori_loop(0, T_CH, body, 0)

    def drain(i, carry):
        pltpu.make_async_copy(rows, acc.at[didx.at[i]], sem).wait()
        return carry

    lax.fori_loop(0, T_CH, drain, 0)
    plsc.subcore_barrier()
    pltpu.sync_copy(acc.at[pl.ds(s * STRIPE, STRIPE)],
                    out_hbm.at[c, pl.ds(s * STRIPE, STRIPE)])


def _tc0_body(x_ref, w_ref, deg_ref, u_ref, dinv_ref):
    deg = deg_ref[0] + deg_ref[1] + 1.0
    rmask = lax.broadcasted_iota(jnp.int32, (RS, 128), 0) < NROW
    dinv = jnp.where(rmask, lax.rsqrt(deg), 0.0)
    z = jnp.dot(x_ref[...], w_ref[...], preferred_element_type=jnp.float32)
    u_ref[...] = dinv * z
    dinv_ref[...] = dinv


_tc0 = pl.pallas_call(
    _tc0_body,
    out_shape=(
        jax.ShapeDtypeStruct((RS, 128), jnp.float32),
        jax.ShapeDtypeStruct((RS, 128), jnp.float32),
    ),
)


def _tc1_body(agg_ref, t_ref, dinv_ref, b_ref, out_ref):
    dinv = dinv_ref[...]
    m = dinv * (agg_ref[0] + agg_ref[1] + t_ref[...])
    h = jnp.maximum(m + b_ref[...], 0.0)
    out_ref[...] = dinv * h


_tc1 = pl.pallas_call(
    _tc1_body,
    out_shape=jax.ShapeDtypeStruct((RS, 128), jnp.float32),
)


def _tcmid_body(agg_ref, t_ref, dinv_ref, w_ref, out_ref):
    dinv = dinv_ref[...]
    m = dinv * (agg_ref[0] + agg_ref[1] + t_ref[...])
    col = lax.broadcasted_iota(jnp.int32, (RS, 128), 1)
    m = jnp.where(col % D == 6, 1.0, m)
    h = jnp.maximum(
        jnp.dot(m, w_ref[...], preferred_element_type=jnp.float32), 0.0)
    out_ref[...] = dinv * h


_tcmid = pl.pallas_call(
    _tcmid_body,
    out_shape=jax.ShapeDtypeStruct((RS, 128), jnp.float32),
)


def _tcfin_body(agg_ref, t_ref, dinv_ref, w_ref, oh_ref, out_ref):
    dinv = dinv_ref[...]
    m = dinv * (agg_ref[0] + agg_ref[1] + t_ref[...])
    col = lax.broadcasted_iota(jnp.int32, (RS, 128), 1)
    m = jnp.where(col % D == 6, 1.0, m)
    h5 = jnp.maximum(
        jnp.dot(m, w_ref[...], preferred_element_type=jnp.float32), 0.0)
    oh = oh_ref[...]
    big = lax.dot_general(h5, oh, (((0,), (0,)), ((), ())),
                          preferred_element_type=jnp.float32)
    csum = jnp.sum(oh, axis=0, keepdims=True)
    sums = jnp.zeros((16, 16), jnp.float32)
    cnts = jnp.zeros((1, 16), jnp.float32)
    for k in range(8):
        sums = sums + big[k * 16:(k + 1) * 16, k * 16:(k + 1) * 16]
        cnts = cnts + csum[:, k * 16:(k + 1) * 16]
    mean_t = sums / jnp.maximum(cnts, 1.0)
    mx = jnp.max(mean_t, axis=0, keepdims=True)
    lse = jnp.log(jnp.sum(jnp.exp(mean_t - mx), axis=0, keepdims=True))
    out_ref[...] = mean_t - mx - lse


_tcfin = pl.pallas_call(
    _tcfin_body,
    out_shape=jax.ShapeDtypeStruct((16, 16), jnp.float32),
)


def _stack_w(W, b):
    Ws = jnp.zeros((D, D), jnp.float32)
    Ws = Ws.at[:W.shape[0], :W.shape[1]].set(W)
    return Ws.at[6, :b.shape[0]].set(b)


def _kron8(Ws):
    return jnp.kron(jnp.eye(8, dtype=jnp.float32), Ws)


def kernel(x, edge_index, batch, W1, b1, W2, b2, W3, b3, W4, b4, Wf, bf):
    f32 = jnp.float32
    src = edge_index[0]
    dst = edge_index[1]
    pad = EP - E
    pad_idx = N + (jnp.arange(pad, dtype=jnp.int32) % (NT - N))
    src2d = jnp.concatenate([src, pad_idx]).reshape(EP // CB, CB)
    dst2d = jnp.concatenate([dst, pad_idx]).reshape(EP // CB, CB)

    zeros_tab = jnp.zeros((NT, D), f32)
    ones_cb = jnp.ones((CB, D), f32)

    x_rs = jnp.zeros((NT, F), f32).at[:N].set(x).reshape(RS, 8 * F)
    w1big = jnp.kron(jnp.eye(8, dtype=f32),
                     jnp.pad(W1, ((0, 0), (0, D - W1.shape[1]))))
    b1bc = jnp.tile(jnp.pad(b1, (0, D - b1.shape[0])), 8)[None, :]

    batch_pad = jnp.concatenate(
        [batch, jnp.full((NT - N,), -1, jnp.int32)])
    oh_rs = (batch_pad[:, None] == jnp.arange(16)[None, :]).astype(
        f32).reshape(RS, 128)

    deg2 = _sc_deg(ones_cb, dst2d, zeros_tab)
    u1, dinv = _tc0(x_rs, w1big, deg2.reshape(NC, RS, 128))

    s = _sc_agg(u1.reshape(NT, D), src2d, dst2d, zeros_tab)
    t = _tc1(s.reshape(NC, RS, 128), u1, dinv, b1bc)

    for Wl, bl in ((W2, b2), (W3, b3), (W4, b4)):
        s = _sc_agg(t.reshape(NT, D), src2d, dst2d, zeros_tab)
        t = _tcmid(s.reshape(NC, RS, 128), t, dinv, _kron8(_stack_w(Wl, bl)))

    s = _sc_agg(t.reshape(NT, D), src2d, dst2d, zeros_tab)
    out_t = _tcfin(s.reshape(NC, RS, 128), t, dinv,
                   _kron8(_stack_w(Wf, bf)), oh_rs)
    return out_t.T


# speedup vs baseline: 91.2567x; 1.5559x over previous
import functools

import jax
import jax.numpy as jnp
import numpy as np
from jax import lax
from jax.experimental import pallas as pl
from jax.experimental.pallas import tpu as pltpu
from jax.experimental.pallas import tpu_sc as plsc

N = 10000
E = 320000
F = 128
NT = 10112
D = 16
CB = 128
NC, NS = 2, 16
NW = NC * NS
T_CH = 80
EP = NW * T_CH * CB
RS = NT // 8
NROW = N // 8
STRIPE = NT // NS


def _sc_mesh():
    return plsc.VectorSubcoreMesh(core_axis_name="c", subcore_axis_name="s")


NSLOT = 8
LA = NSLOT // 2


@functools.partial(
    pl.kernel,
    out_type=jax.ShapeDtypeStruct((NC, NT, D), jnp.float32),
    mesh=_sc_mesh(),
    compiler_params=pltpu.CompilerParams(use_tc_tiling_on_sc=False),
    scratch_types=[
        pltpu.VMEM((T_CH, CB), jnp.int32),
        pltpu.VMEM((T_CH, CB), jnp.int32),
        pltpu.VMEM((NSLOT, CB, D), jnp.float32),
        pltpu.VMEM_SHARED((NT, D), jnp.float32),
        pltpu.VMEM_SHARED((NT, D), jnp.float32),
    ] + [pltpu.SemaphoreType.DMA] * (2 * NSLOT),
)
def _sc_agg(tab_hbm, src_hbm, dst_hbm, zeros_hbm, out_hbm,
            sidx, didx, rows, tab_sh, acc, *sems):
    gsem = sems[:NSLOT]
    ssem = sems[NSLOT:]
    c = lax.axis_index("c")
    s = lax.axis_index("s")
    wid = c * NS + s
    pltpu.sync_copy(zeros_hbm.at[pl.ds(s * STRIPE, STRIPE)],
                    acc.at[pl.ds(s * STRIPE, STRIPE)])
    pltpu.sync_copy(tab_hbm.at[pl.ds(s * STRIPE, STRIPE)],
                    tab_sh.at[pl.ds(s * STRIPE, STRIPE)])
    pltpu.sync_copy(src_hbm.at[pl.ds(wid * T_CH, T_CH)], sidx)
    pltpu.sync_copy(dst_hbm.at[pl.ds(wid * T_CH, T_CH)], didx)
    plsc.subcore_barrier()

    def g_start(slot, i):
        pltpu.async_copy(tab_sh.at[sidx.at[i]], rows.at[slot], gsem[slot])

    def g_wait(slot, i):
        pltpu.make_async_copy(
            tab_sh.at[sidx.at[i]], rows.at[slot], gsem[slot]).wait()

    def s_start(slot, i):
        pltpu.async_copy(rows.at[slot], acc.at[didx.at[i]], ssem[slot],
                         add=True)

    def s_wait(slot, i):
        pltpu.make_async_copy(
            rows.at[slot], acc.at[didx.at[i]], ssem[slot]).wait()

    for j in range(LA):
        g_start(j, j)

    def body(k, carry):
        for b in range(NSLOT):
            i = k * NSLOT + b
            g_wait(b, i)
            s_start(b, i)
            nxt = (b + LA) % NSLOT

            @pl.when(i >= LA)
            def _():
                s_wait(nxt, i - LA)

            @pl.when(i + LA < T_CH)
            def _():
                g_start(nxt, i + LA)
        return carry

    lax.fori_loop(0, T_CH // NSLOT, body, 0)
    for j in range(LA):
        i = T_CH - LA + j
        s_wait(i % NSLOT, i)
    plsc.subcore_barrier()
    pltpu.sync_copy(acc.at[pl.ds(s * STRIPE, STRIPE)],
                    out_hbm.at[c, pl.ds(s * STRIPE, STRIPE)])


@functools.partial(
    pl.kernel,
    out_type=jax.ShapeDtypeStruct((NC, NT, D), jnp.float32),
    mesh=_sc_mesh(),
    compiler_params=pltpu.CompilerParams(use_tc_tiling_on_sc=False),
    scratch_types=[
        pltpu.VMEM((T_CH, CB), jnp.int32),
        pltpu.VMEM((CB, D), jnp.float32),
        pltpu.VMEM_SHARED((NT, D), jnp.float32),
        pltpu.SemaphoreType.DMA,
    ],
)
def _sc_deg(ones_hbm, dst_hbm, zeros_hbm, out_hbm, didx, rows, acc, sem):
    c = lax.axis_index("c")
    s = lax.axis_index("s")
    wid = c * NS + s
    pltpu.sync_copy(zeros_hbm.at[pl.ds(s * STRIPE, STRIPE)],
                    acc.at[pl.ds(s * STRIPE, STRIPE)])
    pltpu.sync_copy(dst_hbm.at[pl.ds(wid * T_CH, T_CH)], didx)
    pltpu.sync_copy(ones_hbm, rows)
    plsc.subcore_barrier()

    def body(i, carry):
        pltpu.async_copy(rows, acc.at[didx.at[i]], sem, add=True)
        return carry

    lax.fori_loop(0, T_CH, body, 0)

    def drain(i, carry):
        pltpu.make_async_copy(rows, acc.at[didx.at[i]], sem).wait()
        return carry

    lax.fori_loop(0, T_CH, drain, 0)
    plsc.subcore_barrier()
    pltpu.sync_copy(acc.at[pl.ds(s * STRIPE, STRIPE)],
                    out_hbm.at[c, pl.ds(s * STRIPE, STRIPE)])


def _tc0_body(x_ref, w_ref, deg_ref, u_ref, dinv_ref):
    deg = deg_ref[0] + deg_ref[1] + 1.0
    rmask = lax.broadcasted_iota(jnp.int32, (RS, 128), 0) < NROW
    dinv = jnp.where(rmask, lax.rsqrt(deg), 0.0)
    z = jnp.dot(x_ref[...], w_ref[...], preferred_element_type=jnp.float32)
    u_ref[...] = dinv * z
    dinv_ref[...] = dinv


_tc0 = pl.pallas_call(
    _tc0_body,
    out_shape=(
        jax.ShapeDtypeStruct((RS, 128), jnp.float32),
        jax.ShapeDtypeStruct((RS, 128), jnp.float32),
    ),
)


def _tc1_body(agg_ref, t_ref, dinv_ref, b_ref, out_ref):
    dinv = dinv_ref[...]
    m = dinv * (agg_ref[0] + agg_ref[1] + t_ref[...])
    h = jnp.maximum(m + b_ref[...], 0.0)
    out_ref[...] = dinv * h


_tc1 = pl.pallas_call(
    _tc1_body,
    out_shape=jax.ShapeDtypeStruct((RS, 128), jnp.float32),
)


def _tcmid_body(agg_ref, t_ref, dinv_ref, w_ref, out_ref):
    dinv = dinv_ref[...]
    m = dinv * (agg_ref[0] + agg_ref[1] + t_ref[...])
    col = lax.broadcasted_iota(jnp.int32, (RS, 128), 1)
    m = jnp.where(col % D == 6, 1.0, m)
    h = jnp.maximum(
        jnp.dot(m, w_ref[...], preferred_element_type=jnp.float32), 0.0)
    out_ref[...] = dinv * h


_tcmid = pl.pallas_call(
    _tcmid_body,
    out_shape=jax.ShapeDtypeStruct((RS, 128), jnp.float32),
)


def _tcfin_body(agg_ref, t_ref, dinv_ref, w_ref, oh_ref, out_ref):
    dinv = dinv_ref[...]
    m = dinv * (agg_ref[0] + agg_ref[1] + t_ref[...])
    col = lax.broadcasted_iota(jnp.int32, (RS, 128), 1)
    m = jnp.where(col % D == 6, 1.0, m)
    h5 = jnp.maximum(
        jnp.dot(m, w_ref[...], preferred_element_type=jnp.float32), 0.0)
    oh = oh_ref[...]
    big = lax.dot_general(h5, oh, (((0,), (0,)), ((), ())),
                          preferred_element_type=jnp.float32)
    csum = jnp.sum(oh, axis=0, keepdims=True)
    sums = jnp.zeros((16, 16), jnp.float32)
    cnts = jnp.zeros((1, 16), jnp.float32)
    for k in range(8):
        sums = sums + big[k * 16:(k + 1) * 16, k * 16:(k + 1) * 16]
        cnts = cnts + csum[:, k * 16:(k + 1) * 16]
    mean_t = sums / jnp.maximum(cnts, 1.0)
    mx = jnp.max(mean_t, axis=0, keepdims=True)
    lse = jnp.log(jnp.sum(jnp.exp(mean_t - mx), axis=0, keepdims=True))
    out_ref[...] = mean_t - mx - lse


_tcfin = pl.pallas_call(
    _tcfin_body,
    out_shape=jax.ShapeDtypeStruct((16, 16), jnp.float32),
)


def _stack_w(W, b):
    Ws = jnp.zeros((D, D), jnp.float32)
    Ws = Ws.at[:W.shape[0], :W.shape[1]].set(W)
    return Ws.at[6, :b.shape[0]].set(b)


def _kron8(Ws):
    return jnp.kron(jnp.eye(8, dtype=jnp.float32), Ws)


def kernel(x, edge_index, batch, W1, b1, W2, b2, W3, b3, W4, b4, Wf, bf):
    f32 = jnp.float32
    src = edge_index[0]
    dst = edge_index[1]
    pad = EP - E
    pad_idx = N + (jnp.arange(pad, dtype=jnp.int32) % (NT - N))
    src2d = jnp.concatenate([src, pad_idx]).reshape(EP // CB, CB)
    dst2d = jnp.concatenate([dst, pad_idx]).reshape(EP // CB, CB)

    zeros_tab = jnp.zeros((NT, D), f32)
    ones_cb = jnp.ones((CB, D), f32)

    x_rs = jnp.zeros((NT, F), f32).at[:N].set(x).reshape(RS, 8 * F)
    w1big = jnp.kron(jnp.eye(8, dtype=f32),
                     jnp.pad(W1, ((0, 0), (0, D - W1.shape[1]))))
    b1bc = jnp.tile(jnp.pad(b1, (0, D - b1.shape[0])), 8)[None, :]

    batch_pad = jnp.concatenate(
        [batch, jnp.full((NT - N,), -1, jnp.int32)])
    oh_rs = (batch_pad[:, None] == jnp.arange(16)[None, :]).astype(
        f32).reshape(RS, 128)

    deg2 = _sc_deg(ones_cb, dst2d, zeros_tab)
    u1, dinv = _tc0(x_rs, w1big, deg2.reshape(NC, RS, 128))

    s = _sc_agg(u1.reshape(NT, D), src2d, dst2d, zeros_tab)
    t = _tc1(s.reshape(NC, RS, 128), u1, dinv, b1bc)

    for Wl, bl in ((W2, b2), (W3, b3), (W4, b4)):
        s = _sc_agg(t.reshape(NT, D), src2d, dst2d, zeros_tab)
        t = _tcmid(s.reshape(NC, RS, 128), t, dinv, _kron8(_stack_w(Wl, bl)))

    s = _sc_agg(t.reshape(NT, D), src2d, dst2d, zeros_tab)
    out_t = _tcfin(s.reshape(NC, RS, 128), t, dinv,
                   _kron8(_stack_w(Wf, bf)), oh_rs)
    return out_t.T


# speedup vs baseline: 120.1669x; 1.3168x over previous
import functools

import jax
import jax.numpy as jnp
import numpy as np
from jax import lax
from jax.experimental import pallas as pl
from jax.experimental.pallas import tpu as pltpu
from jax.experimental.pallas import tpu_sc as plsc

N = 10000
E = 320000
F = 128
NT = 10112
D = 8
CB = 128
NC, NS = 2, 16
NW = NC * NS
T_CH = 80
EP = NW * T_CH * CB
PACK = 128 // D
RS = NT // PACK
NROW = N // PACK
STRIPE = NT // NS


def _sc_mesh():
    return plsc.VectorSubcoreMesh(core_axis_name="c", subcore_axis_name="s")


NSLOT = 8
LA = NSLOT // 2


@functools.partial(
    pl.kernel,
    out_type=jax.ShapeDtypeStruct((NC, NT, D), jnp.float32),
    mesh=_sc_mesh(),
    compiler_params=pltpu.CompilerParams(use_tc_tiling_on_sc=False),
    scratch_types=[
        pltpu.VMEM((T_CH, CB), jnp.int32),
        pltpu.VMEM((T_CH, CB), jnp.int32),
        pltpu.VMEM((NSLOT, CB, D), jnp.float32),
        pltpu.VMEM_SHARED((NT, D), jnp.float32),
        pltpu.VMEM_SHARED((NT, D), jnp.float32),
    ] + [pltpu.SemaphoreType.DMA] * (2 * NSLOT),
)
def _sc_agg(tab_hbm, src_hbm, dst_hbm, zeros_hbm, out_hbm,
            sidx, didx, rows, tab_sh, acc, *sems):
    gsem = sems[:NSLOT]
    ssem = sems[NSLOT:]
    c = lax.axis_index("c")
    s = lax.axis_index("s")
    wid = c * NS + s
    pltpu.sync_copy(zeros_hbm.at[pl.ds(s * STRIPE, STRIPE)],
                    acc.at[pl.ds(s * STRIPE, STRIPE)])
    pltpu.sync_copy(tab_hbm.at[pl.ds(s * STRIPE, STRIPE)],
                    tab_sh.at[pl.ds(s * STRIPE, STRIPE)])
    pltpu.sync_copy(src_hbm.at[pl.ds(wid * T_CH, T_CH)], sidx)
    pltpu.sync_copy(dst_hbm.at[pl.ds(wid * T_CH, T_CH)], didx)
    plsc.subcore_barrier()

    def g_start(slot, i):
        pltpu.async_copy(tab_sh.at[sidx.at[i]], rows.at[slot], gsem[slot])

    def g_wait(slot, i):
        pltpu.make_async_copy(
            tab_sh.at[sidx.at[i]], rows.at[slot], gsem[slot]).wait()

    def s_start(slot, i):
        pltpu.async_copy(rows.at[slot], acc.at[didx.at[i]], ssem[slot],
                         add=True)

    def s_wait(slot, i):
        pltpu.make_async_copy(
            rows.at[slot], acc.at[didx.at[i]], ssem[slot]).wait()

    for j in range(LA):
        g_start(j, j)

    def body(k, carry):
        for b in range(NSLOT):
            i = k * NSLOT + b
            g_wait(b, i)
            s_start(b, i)
            nxt = (b + LA) % NSLOT

            @pl.when(i >= LA)
            def _():
                s_wait(nxt, i - LA)

            @pl.when(i + LA < T_CH)
            def _():
                g_start(nxt, i + LA)
        return carry

    lax.fori_loop(0, T_CH // NSLOT, body, 0)
    for j in range(LA):
        i = T_CH - LA + j
        s_wait(i % NSLOT, i)
    plsc.subcore_barrier()
    pltpu.sync_copy(acc.at[pl.ds(s * STRIPE, STRIPE)],
                    out_hbm.at[c, pl.ds(s * STRIPE, STRIPE)])


@functools.partial(
    pl.kernel,
    out_type=jax.ShapeDtypeStruct((NC, NT, D), jnp.float32),
    mesh=_sc_mesh(),
    compiler_params=pltpu.CompilerParams(use_tc_tiling_on_sc=False),
    scratch_types=[
        pltpu.VMEM((T_CH, CB), jnp.int32),
        pltpu.VMEM((CB, D), jnp.float32),
        pltpu.VMEM_SHARED((NT, D), jnp.float32),
        pltpu.SemaphoreType.DMA,
    ],
)
def _sc_deg(ones_hbm, dst_hbm, zeros_hbm, out_hbm, didx, rows, acc, sem):
    c = lax.axis_index("c")
    s = lax.axis_index("s")
    wid = c * NS + s
    pltpu.sync_copy(zeros_hbm.at[pl.ds(s * STRIPE, STRIPE)],
                    acc.at[pl.ds(s * STRIPE, STRIPE)])
    pltpu.sync_copy(dst_hbm.at[pl.ds(wid * T_CH, T_CH)], didx)
    pltpu.sync_copy(ones_hbm, rows)
    plsc.subcore_barrier()

    def body(i, carry):
        pltpu.async_copy(rows, acc.at[didx.at[i]], sem, add=True)
        return carry

    lax.fori_loop(0, T_CH, body, 0)

    def drain(i, carry):
        pltpu.make_async_copy(rows, acc.at[didx.at[i]], sem).wait()
        return carry

    lax.fori_loop(0, T_CH, drain, 0)
    plsc.subcore_barrier()
    pltpu.sync_copy(acc.at[pl.ds(s * STRIPE, STRIPE)],
                    out_hbm.at[c, pl.ds(s * STRIPE, STRIPE)])


def _tc0_body(x_ref, w_ref, deg_ref, u_ref, dinv_ref):
    deg = deg_ref[0] + deg_ref[1] + 1.0
    rmask = lax.broadcasted_iota(jnp.int32, (RS, 128), 0) < NROW
    dinv = jnp.where(rmask, lax.rsqrt(deg), 0.0)
    z = jnp.dot(x_ref[...], w_ref[...], preferred_element_type=jnp.float32)
    u_ref[...] = dinv * z
    dinv_ref[...] = dinv


_tc0 = pl.pallas_call(
    _tc0_body,
    out_shape=(
        jax.ShapeDtypeStruct((RS, 128), jnp.float32),
        jax.ShapeDtypeStruct((RS, 128), jnp.float32),
    ),
)


def _tc1_body(agg_ref, t_ref, dinv_ref, b_ref, out_ref):
    dinv = dinv_ref[...]
    m = dinv * (agg_ref[0] + agg_ref[1] + t_ref[...])
    h = jnp.maximum(m + b_ref[...], 0.0)
    out_ref[...] = dinv * h


_tc1 = pl.pallas_call(
    _tc1_body,
    out_shape=jax.ShapeDtypeStruct((RS, 128), jnp.float32),
)


def _tcmid_body(agg_ref, t_ref, dinv_ref, w_ref, out_ref):
    dinv = dinv_ref[...]
    m = dinv * (agg_ref[0] + agg_ref[1] + t_ref[...])
    col = lax.broadcasted_iota(jnp.int32, (RS, 128), 1)
    m = jnp.where(col % D == 6, 1.0, m)
    h = jnp.maximum(
        jnp.dot(m, w_ref[...], preferred_element_type=jnp.float32), 0.0)
    out_ref[...] = dinv * h


_tcmid = pl.pallas_call(
    _tcmid_body,
    out_shape=jax.ShapeDtypeStruct((RS, 128), jnp.float32),
)


def _tcfin_body(agg_ref, t_ref, dinv_ref, w_ref, oh_ref, out_ref):
    dinv = dinv_ref[...]
    m = dinv * (agg_ref[0] + agg_ref[1] + t_ref[...])
    col = lax.broadcasted_iota(jnp.int32, (RS, 128), 1)
    m = jnp.where(col % D == 6, 1.0, m)
    h5 = jnp.maximum(
        jnp.dot(m, w_ref[...], preferred_element_type=jnp.float32), 0.0)
    oh = oh_ref[...]
    big = lax.dot_general(h5, oh, (((0,), (0,)), ((), ())),
                          preferred_element_type=jnp.float32)
    csum = jnp.sum(oh, axis=0, keepdims=True)
    sums = jnp.zeros((16, 16), jnp.float32)
    cnts = jnp.zeros((1, 16), jnp.float32)
    for k in range(PACK):
        sums = sums + big[k * 16:(k + 1) * 16, k * 16:(k + 1) * 16]
        cnts = cnts + csum[:, k * 16:(k + 1) * 16]
    mean_t = sums / jnp.maximum(cnts, 1.0)
    mx = jnp.max(mean_t, axis=0, keepdims=True)
    lse = jnp.log(jnp.sum(jnp.exp(mean_t - mx), axis=0, keepdims=True))
    out_ref[...] = mean_t - mx - lse


_tcfin = pl.pallas_call(
    _tcfin_body,
    out_shape=jax.ShapeDtypeStruct((16, 16), jnp.float32),
)


def _stack_w(W, b, dout):
    Ws = jnp.zeros((D, dout), jnp.float32)
    Ws = Ws.at[:W.shape[0], :W.shape[1]].set(W)
    return Ws.at[6, :b.shape[0]].set(b)


def _kron(Ws):
    return jnp.kron(jnp.eye(PACK, dtype=jnp.float32), Ws)


def kernel(x, edge_index, batch, W1, b1, W2, b2, W3, b3, W4, b4, Wf, bf):
    f32 = jnp.float32
    src = edge_index[0]
    dst = edge_index[1]
    pad = EP - E
    pad_idx = N + (jnp.arange(pad, dtype=jnp.int32) % (NT - N))
    src2d = jnp.concatenate([src, pad_idx]).reshape(EP // CB, CB)
    dst2d = jnp.concatenate([dst, pad_idx]).reshape(EP // CB, CB)

    zeros_tab = jnp.zeros((NT, D), f32)
    ones_cb = jnp.ones((CB, D), f32)

    x_rs = jnp.zeros((NT, F), f32).at[:N].set(x).reshape(RS, PACK * F)
    w1big = jnp.kron(jnp.eye(PACK, dtype=f32),
                     jnp.pad(W1, ((0, 0), (0, D - W1.shape[1]))))
    b1bc = jnp.tile(jnp.pad(b1, (0, D - b1.shape[0])), PACK)[None, :]

    batch_pad = jnp.concatenate(
        [batch, jnp.full((NT - N,), -1, jnp.int32)])
    oh_rs = (batch_pad[:, None] == jnp.arange(16)[None, :]).astype(
        f32).reshape(RS, PACK * 16)

    deg2 = _sc_deg(ones_cb, dst2d, zeros_tab)
    u1, dinv = _tc0(x_rs, w1big, deg2.reshape(NC, RS, 128))

    s = _sc_agg(u1.reshape(NT, D), src2d, dst2d, zeros_tab)
    t = _tc1(s.reshape(NC, RS, 128), u1, dinv, b1bc)

    for Wl, bl in ((W2, b2), (W3, b3), (W4, b4)):
        s = _sc_agg(t.reshape(NT, D), src2d, dst2d, zeros_tab)
        t = _tcmid(s.reshape(NC, RS, 128), t, dinv,
                   _kron(_stack_w(Wl, bl, D)))

    s = _sc_agg(t.reshape(NT, D), src2d, dst2d, zeros_tab)
    out_t = _tcfin(s.reshape(NC, RS, 128), t, dinv,
                   _kron(_stack_w(Wf, bf, 16)), oh_rs)
    return out_t.T
